# Initial kernel scaffold; baseline (speedup 1.0000x reference)
import jax
import jax.numpy as jnp
from jax.experimental import pallas as pl


def kernel(inputs, W, b):
    raise NotImplementedError("write your pallas kernel here")


# speedup vs baseline: 40.1227x; 40.1227x over previous
import functools

import jax
import jax.numpy as jnp
from jax.experimental import pallas as pl
from jax.experimental.pallas import tpu as pltpu

UNITS = 4096
K = 409
INPUT_DIM = 256
BATCH_BLOCK = 512
N_SEARCH_ITERS = 31


def _wta_kernel(x_ref, w_ref, b_ref, o_ref):
    x = x_ref[...]
    w = w_ref[...]
    b = b_ref[...]
    out = jnp.dot(x, w, preferred_element_type=jnp.float32) + b
    out = jnp.maximum(out, 0.0)

    bits = jax.lax.bitcast_convert_type(out, jnp.int32)

    bb = out.shape[0]
    lo = jnp.zeros((bb, 1), jnp.int32)
    hi = jnp.max(bits, axis=1, keepdims=True)

    def body(_, carry):
        lo, hi = carry
        mid = lo + ((hi - lo + 1) >> 1)
        cnt = jnp.sum((bits >= mid).astype(jnp.int32), axis=1, keepdims=True)
        ge = cnt >= K
        lo = jnp.where(ge, mid, lo)
        hi = jnp.where(ge, hi, mid - 1)
        return lo, hi

    lo, _ = jax.lax.fori_loop(0, N_SEARCH_ITERS, body, (lo, hi))
    o_ref[...] = jnp.where(bits >= lo, out, 0.0)


@jax.jit
def kernel(inputs, W, b):
    batch = inputs.shape[0]
    grid = (batch // BATCH_BLOCK,)
    b2 = b.reshape(1, UNITS)
    return pl.pallas_call(
        _wta_kernel,
        grid=grid,
        in_specs=[
            pl.BlockSpec((BATCH_BLOCK, INPUT_DIM), lambda i: (i, 0)),
            pl.BlockSpec((INPUT_DIM, UNITS), lambda i: (0, 0)),
            pl.BlockSpec((1, UNITS), lambda i: (0, 0)),
        ],
        out_specs=pl.BlockSpec((BATCH_BLOCK, UNITS), lambda i: (i, 0)),
        out_shape=jax.ShapeDtypeStruct((batch, UNITS), jnp.float32),
        compiler_params=pltpu.CompilerParams(
            dimension_semantics=("arbitrary",),
        ),
    )(inputs, W, b2)
